# baseline (speedup 1.0000x reference)
import functools

import jax
import jax.numpy as jnp
from jax import lax
from jax.experimental import pallas as pl
from jax.experimental.pallas import tpu as pltpu
from jax.experimental.pallas import tpu_sc as plsc

GRID_RES = 64
FOV = 10.0
N = 8192 * 512
NW = 32
PER_W = N // NW
H = 2048
C = 2 * H
NCHUNK = PER_W // C
L = 16
GRID_N = GRID_RES ** 3
FMAX = 3.4028235e38
DEBIAS = -8322.0
BIAS = 33.5


def _tri_kernel(x_hbm, y_hbm, z_hbm, r_hbm, grid_hbm, out_hbm,
                grid_sh, rv, xv, yv, zv, wv, ov,
                ia0, ia1, ia2, ia3,
                ib0, ib1, ib2, ib3,
                va0, va1, va2, va3,
                vb0, vb1, vb2, vb3,
                sem_in, sem_a, sem_b):
    cid = lax.axis_index("c")
    sid = lax.axis_index("s")
    wid = cid * 16 + sid

    @pl.when(sid == 0)
    def _():
        pltpu.sync_copy(grid_hbm, grid_sh)
    pltpu.sync_copy(r_hbm, rv)
    plsc.subcore_barrier()

    ias = (ia0, ia1, ia2, ia3)
    ibs = (ib0, ib1, ib2, ib3)
    vas = (va0, va1, va2, va3)
    vbs = (vb0, vb1, vb2, vb3)

    def pass1(i, off, idxr):
        s = pl.ds(off + i * L, L)
        r00 = rv[0, pl.ds(0, L)]
        r01 = rv[1, pl.ds(0, L)]
        r02 = rv[2, pl.ds(0, L)]
        r10 = rv[3, pl.ds(0, L)]
        r11 = rv[4, pl.ds(0, L)]
        r12 = rv[5, pl.ds(0, L)]
        r20 = rv[6, pl.ds(0, L)]
        r21 = rv[7, pl.ds(0, L)]
        r22 = rv[8, pl.ds(0, L)]
        xx = xv[s]
        yy = yv[s]
        zz = zv[s]
        ex = r00 * xx + r01 * yy + r02 * zz + BIAS
        ey = r10 * xx + r11 * yy + r12 * zz + BIAS
        ez = r20 * xx + r21 * yy + r22 * zz + BIAS
        zero = jnp.zeros((L,), jnp.float32)
        mx = jnp.abs(ex) <= FMAX
        my = jnp.abs(ey) <= FMAX
        mz = jnp.abs(ez) <= FMAX
        bias = jnp.full((L,), BIAS, jnp.float32)
        ex = jnp.where(mx, ex, bias)
        ey = jnp.where(my, ey, bias)
        ez = jnp.where(mz, ez, bias)

        def floor_parts(e):
            e = jnp.clip(e, 0.0, 68.0)
            tf = e.astype(jnp.int32).astype(jnp.float32)
            f = e - tf
            w0 = jnp.where(tf >= 2.0, 1.0 - f, zero)
            w0 = jnp.where(tf <= 65.0, w0, zero)
            w1 = jnp.where(tf >= 1.0, f, zero)
            w1 = jnp.where(tf <= 64.0, w1, zero)
            return tf, w0, w1

        tx, wx0, wx1 = floor_parts(ex)
        ty, wy0, wy1 = floor_parts(ey)
        tz, wz0, wz1 = floor_parts(ez)
        ax0 = jnp.clip(tx, 2.0, 65.0)
        ax1 = jnp.clip(tx + 1.0, 2.0, 65.0)
        ay0 = jnp.clip(ty, 2.0, 65.0)
        ay1 = jnp.clip(ty + 1.0, 2.0, 65.0)
        zbb = jnp.clip(tz, 2.0, 64.0)
        uz0 = jnp.where(tz == 1.0, wz1, jnp.where(tz <= 64.0, wz0, zero))
        uz1 = jnp.where(tz == 1.0, zero, jnp.where(tz == 65.0, wz0, wz1))
        wx0 = jnp.where(mx, wx0, zero)
        wx1 = jnp.where(mx, wx1, zero)

        bx0 = ax0 * 4096.0 + DEBIAS
        bx1 = ax1 * 4096.0 + DEBIAS
        by0 = ay0 * 64.0
        by1 = ay1 * 64.0
        si = pl.ds(i * L, L)
        idxr[0][si] = (bx0 + by0 + zbb).astype(jnp.int32)
        idxr[1][si] = (bx0 + by1 + zbb).astype(jnp.int32)
        idxr[2][si] = (bx1 + by0 + zbb).astype(jnp.int32)
        idxr[3][si] = (bx1 + by1 + zbb).astype(jnp.int32)
        wv[0, s] = wx0 * wy0
        wv[1, s] = wx0 * wy1
        wv[2, s] = wx1 * wy0
        wv[3, s] = wx1 * wy1
        wv[4, s] = uz0
        wv[5, s] = uz1

    def pass2(i, off, valr):
        s = pl.ds(off + i * L, L)
        si = pl.ds(i * L, L)
        a00 = wv[0, s]
        a01 = wv[1, s]
        a10 = wv[2, s]
        a11 = wv[3, s]
        uz0 = wv[4, s]
        uz1 = wv[5, s]
        himask = jnp.full((L,), -65536, jnp.int32)

        def unpack2(w):
            lo = jax.lax.bitcast_convert_type(w << 16, jnp.float32)
            hi = jax.lax.bitcast_convert_type(w & himask, jnp.float32)
            return lo, hi

        l0, h0 = unpack2(valr[0][si])
        l1, h1 = unpack2(valr[1][si])
        l2, h2 = unpack2(valr[2][si])
        l3, h3 = unpack2(valr[3][si])
        acc = (a00 * (uz0 * l0 + uz1 * h0)
               + a01 * (uz0 * l1 + uz1 * h1)
               + a10 * (uz0 * l2 + uz1 * h2)
               + a11 * (uz0 * l3 + uz1 * h3))
        ov[s] = acc

    def start_in(base, off):
        cps = [pltpu.make_async_copy(x_hbm.at[pl.ds(base, H)],
                                     xv.at[pl.ds(off, H)], sem_in),
               pltpu.make_async_copy(y_hbm.at[pl.ds(base, H)],
                                     yv.at[pl.ds(off, H)], sem_in),
               pltpu.make_async_copy(z_hbm.at[pl.ds(base, H)],
                                     zv.at[pl.ds(off, H)], sem_in)]
        for cp in cps:
            cp.start()
        return cps

    def wait_in(base, off):
        cps = [pltpu.make_async_copy(x_hbm.at[pl.ds(base, H)],
                                     xv.at[pl.ds(off, H)], sem_in),
               pltpu.make_async_copy(y_hbm.at[pl.ds(base, H)],
                                     yv.at[pl.ds(off, H)], sem_in),
               pltpu.make_async_copy(z_hbm.at[pl.ds(base, H)],
                                     zv.at[pl.ds(off, H)], sem_in)]
        for cp in cps:
            cp.wait()

    def gathers(idxr, valr, sem):
        return [pltpu.make_async_copy(grid_sh.at[ivr], vvr, sem)
                for ivr, vvr in zip(idxr, valr)]

    tile_base = wid * PER_W
    start_in(tile_base, 0)

    def body(k, _):
        base_a = tile_base + (2 * k) * H
        base_b = base_a + H
        wait_in(base_a, 0)
        start_in(base_b, H)

        @plsc.parallel_loop(0, H // L, unroll=4)
        def p1a(i):
            pass1(i, 0, ias)
        for cp in gathers(ias, vas, sem_a):
            cp.start()

        @pl.when(k > 0)
        def _():
            for cp in gathers(ibs, vbs, sem_b):
                cp.wait()

            @plsc.parallel_loop(0, H // L, unroll=4)
            def p2b(i):
                pass2(i, H, vbs)
            pltpu.sync_copy(ov.at[pl.ds(H, H)],
                            out_hbm.at[pl.ds(base_a - H, H)])

        wait_in(base_b, H)
        base_next = jnp.minimum(base_b + H, N - H)
        start_in(base_next, 0)

        @plsc.parallel_loop(0, H // L, unroll=4)
        def p1b(i):
            pass1(i, H, ibs)
        for cp in gathers(ibs, vbs, sem_b):
            cp.start()

        for cp in gathers(ias, vas, sem_a):
            cp.wait()

        @plsc.parallel_loop(0, H // L, unroll=4)
        def p2a(i):
            pass2(i, 0, vas)
        pltpu.sync_copy(ov.at[pl.ds(0, H)], out_hbm.at[pl.ds(base_a, H)])
        return ()

    lax.fori_loop(0, NCHUNK, body, ())

    for cp in gathers(ibs, vbs, sem_b):
        cp.wait()

    @plsc.parallel_loop(0, H // L, unroll=4)
    def p2b_tail(i):
        pass2(i, H, vbs)
    pltpu.sync_copy(ov.at[pl.ds(H, H)],
                    out_hbm.at[pl.ds(tile_base + PER_W - H, H)])
    wait_in(jnp.minimum(tile_base + PER_W, N - H), 0)


@jax.jit
def _run(xf, yf, zf, rarr, gridf):
    mesh = plsc.VectorSubcoreMesh(core_axis_name="c", subcore_axis_name="s")
    f = functools.partial(
        pl.kernel,
        out_type=jax.ShapeDtypeStruct((N,), jnp.float32),
        name="trilinear_grid_sample_sc",
        mesh=mesh,
        scratch_types=(
            [pltpu.VMEM_SHARED((GRID_N,), jnp.int32),
             pltpu.VMEM((9, 16), jnp.float32),
             pltpu.VMEM((C,), jnp.float32),
             pltpu.VMEM((C,), jnp.float32),
             pltpu.VMEM((C,), jnp.float32),
             pltpu.VMEM((6, C), jnp.float32),
             pltpu.VMEM((C,), jnp.float32)]
            + [pltpu.VMEM((H,), jnp.int32)] * 8
            + [pltpu.VMEM((H,), jnp.int32)] * 8
            + [pltpu.SemaphoreType.DMA] * 3
        ),
    )(_tri_kernel)
    return f(xf, yf, zf, rarr, gridf)


def kernel(x, y, z, t, v, axis, grid):
    a = axis / jnp.linalg.norm(axis)
    theta = (-2.0 * jnp.pi) * v * t
    c = jnp.cos(theta)[0]
    s = jnp.sin(theta)[0]
    ax, ay, az = a[0], a[1], a[2]
    omc = 1.0 - c
    scale = (GRID_RES - 1.0) / FOV
    r = jnp.stack([
        c + omc * ax * ax, omc * ax * ay - s * az, omc * ax * az + s * ay,
        omc * ax * ay + s * az, c + omc * ay * ay, omc * ay * az - s * ax,
        omc * ax * az - s * ay, omc * ay * az + s * ax, c + omc * az * az,
    ]) * scale
    rarr = jnp.tile(r.astype(jnp.float32)[:, None], (1, 16))
    gb = grid.reshape(GRID_N).astype(jnp.bfloat16)
    gb1 = jnp.roll(gb, -1)
    packed = jax.lax.bitcast_convert_type(
        jnp.stack([gb, gb1], axis=-1), jnp.int32)
    out = _run(x.reshape(N), y.reshape(N), z.reshape(N), rarr, packed)
    return out.reshape(x.shape)


# speedup vs baseline: 1.0043x; 1.0043x over previous
import functools

import jax
import jax.numpy as jnp
from jax import lax
from jax.experimental import pallas as pl
from jax.experimental.pallas import tpu as pltpu
from jax.experimental.pallas import tpu_sc as plsc

GRID_RES = 64
FOV = 10.0
N = 8192 * 512
NW = 32
PER_W = N // NW
H = 1024
C = 2 * H
NCHUNK = PER_W // C
L = 16
GRID_N = GRID_RES ** 3
FMAX = 3.4028235e38
DEBIAS = -8322.0
BIAS = 33.5


def _tri_kernel(x_hbm, y_hbm, z_hbm, r_hbm, grid_hbm, out_hbm,
                grid_sh, rv, xv, yv, zv, wv, ov,
                ia0, ia1, ia2, ia3,
                ib0, ib1, ib2, ib3,
                va0, va1, va2, va3,
                vb0, vb1, vb2, vb3,
                sem_in, sem_a, sem_b):
    cid = lax.axis_index("c")
    sid = lax.axis_index("s")
    wid = cid * 16 + sid

    @pl.when(sid == 0)
    def _():
        pltpu.sync_copy(grid_hbm, grid_sh)
    pltpu.sync_copy(r_hbm, rv)
    plsc.subcore_barrier()

    ias = (ia0, ia1, ia2, ia3)
    ibs = (ib0, ib1, ib2, ib3)
    vas = (va0, va1, va2, va3)
    vbs = (vb0, vb1, vb2, vb3)

    def pass1(i, off, idxr):
        s = pl.ds(off + i * L, L)
        r00 = rv[0, pl.ds(0, L)]
        r01 = rv[1, pl.ds(0, L)]
        r02 = rv[2, pl.ds(0, L)]
        r10 = rv[3, pl.ds(0, L)]
        r11 = rv[4, pl.ds(0, L)]
        r12 = rv[5, pl.ds(0, L)]
        r20 = rv[6, pl.ds(0, L)]
        r21 = rv[7, pl.ds(0, L)]
        r22 = rv[8, pl.ds(0, L)]
        xx = xv[s]
        yy = yv[s]
        zz = zv[s]
        ex = r00 * xx + r01 * yy + r02 * zz + BIAS
        ey = r10 * xx + r11 * yy + r12 * zz + BIAS
        ez = r20 * xx + r21 * yy + r22 * zz + BIAS
        zero = jnp.zeros((L,), jnp.float32)
        mx = jnp.abs(ex) <= FMAX
        my = jnp.abs(ey) <= FMAX
        mz = jnp.abs(ez) <= FMAX
        bias = jnp.full((L,), BIAS, jnp.float32)
        ex = jnp.where(mx, ex, bias)
        ey = jnp.where(my, ey, bias)
        ez = jnp.where(mz, ez, bias)

        def floor_parts(e):
            e = jnp.clip(e, 0.0, 68.0)
            tf = e.astype(jnp.int32).astype(jnp.float32)
            f = e - tf
            w0 = jnp.where(tf >= 2.0, 1.0 - f, zero)
            w0 = jnp.where(tf <= 65.0, w0, zero)
            w1 = jnp.where(tf >= 1.0, f, zero)
            w1 = jnp.where(tf <= 64.0, w1, zero)
            return tf, w0, w1

        tx, wx0, wx1 = floor_parts(ex)
        ty, wy0, wy1 = floor_parts(ey)
        tz, wz0, wz1 = floor_parts(ez)
        ax0 = jnp.clip(tx, 2.0, 65.0)
        ax1 = jnp.clip(tx + 1.0, 2.0, 65.0)
        ay0 = jnp.clip(ty, 2.0, 65.0)
        ay1 = jnp.clip(ty + 1.0, 2.0, 65.0)
        zbb = jnp.clip(tz, 2.0, 64.0)
        uz0 = jnp.where(tz == 1.0, wz1, jnp.where(tz <= 64.0, wz0, zero))
        uz1 = jnp.where(tz == 1.0, zero, jnp.where(tz == 65.0, wz0, wz1))
        wx0 = jnp.where(mx, wx0, zero)
        wx1 = jnp.where(mx, wx1, zero)

        bx0 = ax0 * 4096.0 + DEBIAS
        bx1 = ax1 * 4096.0 + DEBIAS
        by0 = ay0 * 64.0
        by1 = ay1 * 64.0
        si = pl.ds(i * L, L)
        idxr[0][si] = (bx0 + by0 + zbb).astype(jnp.int32)
        idxr[1][si] = (bx0 + by1 + zbb).astype(jnp.int32)
        idxr[2][si] = (bx1 + by0 + zbb).astype(jnp.int32)
        idxr[3][si] = (bx1 + by1 + zbb).astype(jnp.int32)
        wv[0, s] = wx0 * wy0
        wv[1, s] = wx0 * wy1
        wv[2, s] = wx1 * wy0
        wv[3, s] = wx1 * wy1
        wv[4, s] = uz0
        wv[5, s] = uz1

    def pass2(i, off, valr):
        s = pl.ds(off + i * L, L)
        si = pl.ds(i * L, L)
        a00 = wv[0, s]
        a01 = wv[1, s]
        a10 = wv[2, s]
        a11 = wv[3, s]
        uz0 = wv[4, s]
        uz1 = wv[5, s]
        himask = jnp.full((L,), -65536, jnp.int32)

        def unpack2(w):
            lo = jax.lax.bitcast_convert_type(w << 16, jnp.float32)
            hi = jax.lax.bitcast_convert_type(w & himask, jnp.float32)
            return lo, hi

        l0, h0 = unpack2(valr[0][si])
        l1, h1 = unpack2(valr[1][si])
        l2, h2 = unpack2(valr[2][si])
        l3, h3 = unpack2(valr[3][si])
        acc = (a00 * (uz0 * l0 + uz1 * h0)
               + a01 * (uz0 * l1 + uz1 * h1)
               + a10 * (uz0 * l2 + uz1 * h2)
               + a11 * (uz0 * l3 + uz1 * h3))
        ov[s] = acc

    def start_in(base, off):
        cps = [pltpu.make_async_copy(x_hbm.at[pl.ds(base, H)],
                                     xv.at[pl.ds(off, H)], sem_in),
               pltpu.make_async_copy(y_hbm.at[pl.ds(base, H)],
                                     yv.at[pl.ds(off, H)], sem_in),
               pltpu.make_async_copy(z_hbm.at[pl.ds(base, H)],
                                     zv.at[pl.ds(off, H)], sem_in)]
        for cp in cps:
            cp.start()
        return cps

    def wait_in(base, off):
        cps = [pltpu.make_async_copy(x_hbm.at[pl.ds(base, H)],
                                     xv.at[pl.ds(off, H)], sem_in),
               pltpu.make_async_copy(y_hbm.at[pl.ds(base, H)],
                                     yv.at[pl.ds(off, H)], sem_in),
               pltpu.make_async_copy(z_hbm.at[pl.ds(base, H)],
                                     zv.at[pl.ds(off, H)], sem_in)]
        for cp in cps:
            cp.wait()

    def gathers(idxr, valr, sem):
        return [pltpu.make_async_copy(grid_sh.at[ivr], vvr, sem)
                for ivr, vvr in zip(idxr, valr)]

    tile_base = wid * PER_W
    start_in(tile_base, 0)

    def body(k, _):
        base_a = tile_base + (2 * k) * H
        base_b = base_a + H
        wait_in(base_a, 0)
        start_in(base_b, H)

        @plsc.parallel_loop(0, H // L, unroll=4)
        def p1a(i):
            pass1(i, 0, ias)
        for cp in gathers(ias, vas, sem_a):
            cp.start()

        @pl.when(k > 0)
        def _():
            for cp in gathers(ibs, vbs, sem_b):
                cp.wait()

            @plsc.parallel_loop(0, H // L, unroll=4)
            def p2b(i):
                pass2(i, H, vbs)
            pltpu.sync_copy(ov.at[pl.ds(H, H)],
                            out_hbm.at[pl.ds(base_a - H, H)])

        wait_in(base_b, H)
        base_next = jnp.minimum(base_b + H, N - H)
        start_in(base_next, 0)

        @plsc.parallel_loop(0, H // L, unroll=4)
        def p1b(i):
            pass1(i, H, ibs)
        for cp in gathers(ibs, vbs, sem_b):
            cp.start()

        for cp in gathers(ias, vas, sem_a):
            cp.wait()

        @plsc.parallel_loop(0, H // L, unroll=4)
        def p2a(i):
            pass2(i, 0, vas)
        pltpu.sync_copy(ov.at[pl.ds(0, H)], out_hbm.at[pl.ds(base_a, H)])
        return ()

    lax.fori_loop(0, NCHUNK, body, ())

    for cp in gathers(ibs, vbs, sem_b):
        cp.wait()

    @plsc.parallel_loop(0, H // L, unroll=4)
    def p2b_tail(i):
        pass2(i, H, vbs)
    pltpu.sync_copy(ov.at[pl.ds(H, H)],
                    out_hbm.at[pl.ds(tile_base + PER_W - H, H)])
    wait_in(jnp.minimum(tile_base + PER_W, N - H), 0)


@jax.jit
def _run(xf, yf, zf, rarr, gridf):
    mesh = plsc.VectorSubcoreMesh(core_axis_name="c", subcore_axis_name="s")
    f = functools.partial(
        pl.kernel,
        out_type=jax.ShapeDtypeStruct((N,), jnp.float32),
        name="trilinear_grid_sample_sc",
        mesh=mesh,
        scratch_types=(
            [pltpu.VMEM_SHARED((GRID_N,), jnp.int32),
             pltpu.VMEM((9, 16), jnp.float32),
             pltpu.VMEM((C,), jnp.float32),
             pltpu.VMEM((C,), jnp.float32),
             pltpu.VMEM((C,), jnp.float32),
             pltpu.VMEM((6, C), jnp.float32),
             pltpu.VMEM((C,), jnp.float32)]
            + [pltpu.VMEM((H,), jnp.int32)] * 8
            + [pltpu.VMEM((H,), jnp.int32)] * 8
            + [pltpu.SemaphoreType.DMA] * 3
        ),
    )(_tri_kernel)
    return f(xf, yf, zf, rarr, gridf)


def kernel(x, y, z, t, v, axis, grid):
    a = axis / jnp.linalg.norm(axis)
    theta = (-2.0 * jnp.pi) * v * t
    c = jnp.cos(theta)[0]
    s = jnp.sin(theta)[0]
    ax, ay, az = a[0], a[1], a[2]
    omc = 1.0 - c
    scale = (GRID_RES - 1.0) / FOV
    r = jnp.stack([
        c + omc * ax * ax, omc * ax * ay - s * az, omc * ax * az + s * ay,
        omc * ax * ay + s * az, c + omc * ay * ay, omc * ay * az - s * ax,
        omc * ax * az - s * ay, omc * ay * az + s * ax, c + omc * az * az,
    ]) * scale
    rarr = jnp.tile(r.astype(jnp.float32)[:, None], (1, 16))
    gb = grid.reshape(GRID_N).astype(jnp.bfloat16)
    gb1 = jnp.roll(gb, -1)
    packed = jax.lax.bitcast_convert_type(
        jnp.stack([gb, gb1], axis=-1), jnp.int32)
    out = _run(x.reshape(N), y.reshape(N), z.reshape(N), rarr, packed)
    return out.reshape(x.shape)
